# Initial kernel scaffold; baseline (speedup 1.0000x reference)
import functools

import jax
import jax.numpy as jnp
from jax import lax
from jax.experimental import pallas as pl
from jax.experimental.pallas import tpu as pltpu
from jax.experimental.pallas import tpu_sc as plsc

VOCAB = 16
DIM = 64


def _sc_gather(positions, table, *, num_workers, chunk):
    S = positions.shape[0]
    per_w = S // num_workers
    n_chunks = per_w // chunk
    mesh = plsc.VectorSubcoreMesh(core_axis_name="c", subcore_axis_name="s")
    nc = 2

    @functools.partial(
        pl.kernel,
        mesh=mesh,
        out_type=jax.ShapeDtypeStruct((S, DIM), jnp.float32),
        scratch_types=[
            pltpu.VMEM((chunk,), jnp.int32),
            pltpu.VMEM((chunk, DIM), jnp.float32),
            pltpu.SemaphoreType.DMA,
        ],
    )
    def k(pos_hbm, table_hbm, out_hbm, idx_v, rows_v, sem):
        wid = lax.axis_index("s") * nc + lax.axis_index("c")
        base = wid * per_w

        def body(i, _):
            off = base + i * chunk
            pltpu.sync_copy(pos_hbm.at[pl.ds(off, chunk)], idx_v)
            pltpu.async_copy(table_hbm.at[idx_v], rows_v, sem).wait()
            pltpu.sync_copy(rows_v, out_hbm.at[pl.ds(off, chunk)])
            return _

        lax.fori_loop(0, n_chunks, body, 0)

    return k(positions, table)


def kernel(positions, table):
    return _sc_gather(positions.astype(jnp.int32), table,
                      num_workers=32, chunk=800)


# speedup vs baseline: 6.3633x; 6.3633x over previous
import functools

import jax
import jax.numpy as jnp
from jax import lax
from jax.experimental import pallas as pl
from jax.experimental.pallas import tpu as pltpu
from jax.experimental.pallas import tpu_sc as plsc

VOCAB = 16
DIM = 64
NW = 32
PAIRS = 256
IDX_ROWS = PAIRS // 128


def _pair_table(table):
    pa = jnp.broadcast_to(table[:, None, :], (VOCAB, VOCAB, DIM))
    pb = jnp.broadcast_to(table[None, :, :], (VOCAB, VOCAB, DIM))
    return jnp.concatenate([pa, pb], axis=-1).reshape(VOCAB * VOCAB, 2 * DIM)


def _sc_gather(positions, ptable):
    S = positions.shape[0]
    n_pairs = S // 2
    pairs_per_w = n_pairs // NW
    n = pairs_per_w // PAIRS
    assert pairs_per_w % PAIRS == 0 and n >= 4 and n % 2 == 0
    mesh = plsc.VectorSubcoreMesh(core_axis_name="c", subcore_axis_name="s")

    @functools.partial(
        pl.kernel,
        mesh=mesh,
        out_type=jax.ShapeDtypeStruct((n_pairs, 2 * DIM), jnp.float32),
        scratch_types=(
            [pltpu.VMEM_SHARED((VOCAB * VOCAB, 2 * DIM), jnp.float32)]
            + [pltpu.VMEM((2 * PAIRS,), jnp.int32) for _ in range(2)]
            + [pltpu.VMEM((PAIRS, 2 * DIM), jnp.float32) for _ in range(2)]
            + [pltpu.VMEM((128,), jnp.int32) for _ in range(2 * IDX_ROWS)]
            + [pltpu.SemaphoreType.DMA for _ in range(6)]
        ),
    )
    def k(pos_hbm, ptable_hbm, out_hbm, ptable_sh,
          idx0, idx1, rows0, rows1, c00, c01, c10, c11,
          si0, si1, sg0, sg1, so0, so1):
        idx = (idx0, idx1)
        rows = (rows0, rows1)
        combs = ((c00, c01), (c10, c11))
        sem_i = (si0, si1)
        sem_g = (sg0, sg1)
        sem_o = (so0, so1)

        wid = lax.axis_index("s") * 2 + lax.axis_index("c")
        in_base = wid * (2 * pairs_per_w)
        out_base = wid * pairs_per_w
        lane = lax.iota(jnp.int32, 16)
        half = lane % 8
        dnums = lax.GatherDimensionNumbers(
            offset_dims=(), collapsed_slice_dims=(0,), start_index_map=(0,))

        def perm(v, i):
            return lax.gather(v, i[:, None], dnums, (1,),
                              mode=lax.GatherScatterMode.PROMISE_IN_BOUNDS)

        def idx_src(i):
            return pos_hbm.at[pl.ds(in_base + i * (2 * PAIRS), 2 * PAIRS)]

        def start_idx(i, p):
            pltpu.async_copy(idx_src(i), idx[p], sem_i[p])

        def wait_idx(p):
            pltpu.make_async_copy(idx_src(0), idx[p], sem_i[p]).wait()

        def combine(p):
            for m in range(PAIRS // 16):
                v_lo = idx[p][pl.ds(32 * m, 16)]
                v_hi = idx[p][pl.ds(32 * m + 16, 16)]
                c_lo = VOCAB * perm(v_lo, half * 2) + perm(v_lo, half * 2 + 1)
                c_hi = VOCAB * perm(v_hi, half * 2) + perm(v_hi, half * 2 + 1)
                combs[p][m // 8][pl.ds((m % 8) * 16, 16)] = jnp.where(
                    lane < 8, c_lo, c_hi)

        def fire_gathers(p):
            for j in range(IDX_ROWS):
                pltpu.async_copy(ptable_sh.at[combs[p][j]],
                                 rows[p].at[pl.ds(j * 128, 128)], sem_g[p])

        def wait_gathers(p):
            for j in range(IDX_ROWS):
                pltpu.make_async_copy(ptable_sh.at[combs[p][j]],
                                      rows[p].at[pl.ds(j * 128, 128)],
                                      sem_g[p]).wait()

        def start_out(i, p):
            pltpu.async_copy(rows[p],
                             out_hbm.at[pl.ds(out_base + i * PAIRS, PAIRS)],
                             sem_o[p])

        def wait_out(p):
            pltpu.make_async_copy(rows[p],
                                  out_hbm.at[pl.ds(out_base, PAIRS)],
                                  sem_o[p]).wait()

        @pl.when(lax.axis_index("s") == 0)
        def _stage_table():
            pltpu.sync_copy(ptable_hbm, ptable_sh)

        plsc.subcore_barrier()

        pltpu.sync_copy(idx_src(0), idx[0])
        combine(0)
        fire_gathers(0)
        start_idx(1, 1)

        wait_idx(1)
        combine(1)
        start_idx(2, 0)
        wait_gathers(0)
        fire_gathers(1)
        start_out(0, 0)

        def steady(i, p, q):
            wait_idx(q)
            combine(q)
            start_idx(i + 2, p)
            wait_out(q)
            wait_gathers(p)
            fire_gathers(q)
            start_out(i, p)

        def body(t, carry):
            steady(2 * t + 1, 1, 0)
            steady(2 * t + 2, 0, 1)
            return carry

        lax.fori_loop(0, (n - 4) // 2, body, 0)

        steady(n - 3, 1, 0)
        wait_idx(1)
        combine(1)
        wait_out(1)
        wait_gathers(0)
        fire_gathers(1)
        start_out(n - 2, 0)
        wait_out(0)
        wait_gathers(1)
        start_out(n - 1, 1)
        wait_out(1)

    return k(positions, ptable)


def kernel(positions, table):
    out2 = _sc_gather(positions.astype(jnp.int32), _pair_table(table))
    return out2.reshape(positions.shape[0], DIM)
